# Initial kernel scaffold; baseline (speedup 1.0000x reference)
import jax
import jax.numpy as jnp
from jax.experimental import pallas as pl


def kernel(x, w):
    raise NotImplementedError("write your pallas kernel here")


# speedup vs baseline: 1.4580x; 1.4580x over previous
import functools

import jax
import jax.numpy as jnp
from jax import lax
from jax.experimental import pallas as pl
from jax.experimental.pallas import tpu as pltpu
from jax.experimental.pallas import tpu_sc as plsc

_NUM_CORES = 2
_NUM_SUBCORES = 16
_NUM_WORKERS = _NUM_CORES * _NUM_SUBCORES

_ROW = 128
_K = 8


def _gather_call(idx2d, w):
    rows_total, row = idx2d.shape
    vocab, d = w.shape
    rows_per_w = rows_total // _NUM_WORKERS
    n_chunks = rows_per_w // _K

    mesh = plsc.VectorSubcoreMesh(
        core_axis_name="c", subcore_axis_name="s")

    @functools.partial(
        pl.kernel,
        mesh=mesh,
        compiler_params=pltpu.CompilerParams(use_tc_tiling_on_sc=False),
        out_type=jax.ShapeDtypeStruct((rows_total, row, d), jnp.float32),
        scratch_types=[
            pltpu.VMEM((_K, row), jnp.int32),
            pltpu.VMEM((_K, row, d), jnp.float32),
            pltpu.SemaphoreType.DMA,
        ],
    )
    def body(idx_hbm, w_hbm, out_hbm, idx_v, rows_v, sem):
        wid = lax.axis_index("s") * _NUM_CORES + lax.axis_index("c")
        base = wid * rows_per_w

        def chunk(g, _):
            r0 = base + g * _K
            pltpu.sync_copy(idx_hbm.at[pl.ds(r0, _K)], idx_v)
            for j in range(_K):
                pltpu.async_copy(w_hbm.at[idx_v.at[j]], rows_v.at[j], sem)
            for j in range(_K):
                pltpu.make_async_copy(
                    w_hbm.at[idx_v.at[j]], rows_v.at[j], sem).wait()
            pltpu.sync_copy(rows_v, out_hbm.at[pl.ds(r0, _K)])
            return ()

        lax.fori_loop(0, n_chunks, chunk, (), unroll=False)

    return body(idx2d, w)


def kernel(x, w):
    b, s = x.shape
    vocab, d = w.shape
    idx2d = x.reshape(-1, _ROW)
    out = _gather_call(idx2d, w)
    return out.reshape(b, s, d)


# speedup vs baseline: 1.5004x; 1.0291x over previous
import functools

import jax
import jax.numpy as jnp
from jax import lax
from jax.experimental import pallas as pl
from jax.experimental.pallas import tpu as pltpu
from jax.experimental.pallas import tpu_sc as plsc

_NUM_CORES = 2
_NUM_SUBCORES = 16
_NUM_WORKERS = _NUM_CORES * _NUM_SUBCORES

_ROW = 128
_K = 5
_NB = 4


def _gather_call(idx2d, w):
    rows_total, row = idx2d.shape
    vocab, d = w.shape
    rows_per_w = rows_total // _NUM_WORKERS
    n_chunks = rows_per_w // _K
    n_outer = n_chunks // _NB

    mesh = plsc.VectorSubcoreMesh(
        core_axis_name="c", subcore_axis_name="s")

    @functools.partial(
        pl.kernel,
        mesh=mesh,
        compiler_params=pltpu.CompilerParams(use_tc_tiling_on_sc=False),
        out_type=jax.ShapeDtypeStruct((rows_total, row, d), jnp.float32),
        scratch_types=[
            pltpu.VMEM((rows_per_w, row), jnp.int32),
            pltpu.VMEM((_NB, _K, row, d), jnp.float32),
            [pltpu.SemaphoreType.DMA] * _NB,
            [pltpu.SemaphoreType.DMA] * _NB,
        ],
    )
    def body(idx_hbm, w_hbm, out_hbm, idx_v, rows_v, gsems, osems):
        wid = lax.axis_index("s") * _NUM_CORES + lax.axis_index("c")
        base = wid * rows_per_w

        def fire_gathers(g, b):
            for j in range(_K):
                pltpu.async_copy(
                    w_hbm.at[idx_v.at[g * _K + j]],
                    rows_v.at[b].at[j], gsems[b])

        def wait_gathers(b):
            for j in range(_K):
                pltpu.make_async_copy(
                    w_hbm.at[idx_v.at[j]], rows_v.at[b].at[j],
                    gsems[b]).wait()

        def fire_out(g, b):
            pltpu.async_copy(
                rows_v.at[b], out_hbm.at[pl.ds(base + g * _K, _K)],
                osems[b])

        def wait_out(b):
            pltpu.make_async_copy(
                rows_v.at[b], out_hbm.at[pl.ds(base, _K)], osems[b]).wait()

        pltpu.sync_copy(idx_hbm.at[pl.ds(base, rows_per_w)], idx_v)

        for b in range(_NB - 1):
            fire_gathers(b, b)

        for b in range(_NB):
            wait_gathers(b)
            fire_out(b, b)
            if b > 0:
                wait_out((b + _NB - 1) % _NB)
            fire_gathers(b + _NB - 1, (b + _NB - 1) % _NB)

        def outer(gg, _):
            g0 = gg * _NB
            for b in range(_NB):
                wait_gathers(b)
                fire_out(g0 + b, b)
                wait_out((b + _NB - 1) % _NB)
                fire_gathers(g0 + b + _NB - 1, (b + _NB - 1) % _NB)
            return ()

        lax.fori_loop(1, n_outer - 1, outer, (), unroll=False)

        g0 = (n_outer - 1) * _NB
        for b in range(_NB):
            wait_gathers(b)
            fire_out(g0 + b, b)
            if b == 0:
                wait_out(_NB - 1)
                fire_gathers(g0 + _NB - 1, _NB - 1)

        for b in range(_NB):
            wait_out(b)

    return body(idx2d, w)


def kernel(x, w):
    b, s = x.shape
    vocab, d = w.shape
    idx2d = x.reshape(-1, _ROW)
    out = _gather_call(idx2d, w)
    return out.reshape(b, s, d)
